# Initial kernel scaffold; baseline (speedup 1.0000x reference)
import jax
import jax.numpy as jnp
from jax.experimental import pallas as pl


def kernel(xyz, rgb, centers, center_feats, rW1, rb1, rg, rbt, rW2, rb2, mW1, mb1, mg, mbt, mW2, mb2, fW1, fb1, fg, fbt, fW2, fb2):
    raise NotImplementedError("write your pallas kernel here")


# speedup vs baseline: 26.0525x; 26.0525x over previous
import jax
import jax.numpy as jnp
from jax import lax
from jax.experimental import pallas as pl
from jax.experimental.pallas import tpu as pltpu
from jax.experimental.pallas import tpu_sc as plsc

_B, _N, _M = 2, 8192, 2048
_FEATS, _HIDDEN = 64, 128
_ROW = 128


_NBLK_KNN = 1024


def _knn_body_b(base, xyz_ref, ct_ref, idx_ref, wt_ref):
    x = xyz_ref[...]
    ct = ct_ref[...]
    d0 = x[:, 0:1] - ct[0:1, :]
    d1 = x[:, 1:2] - ct[1:2, :]
    d2 = x[:, 2:3] - ct[2:3, :]
    d = d0 * d0 + d1 * d1 + d2 * d2

    iota = lax.broadcasted_iota(jnp.int32, (_NBLK_KNN, _M), 1).astype(jnp.float32)
    inf = jnp.float32(jnp.inf)
    idxs = []
    sqs = []
    for k in range(3):
        mval = jnp.min(d, axis=1, keepdims=True)
        ik = jnp.min(jnp.where(d == mval, iota, jnp.float32(_M)),
                     axis=1, keepdims=True)
        idxs.append(ik.astype(jnp.int32))
        sqs.append(mval)
        if k < 2:
            d = jnp.where(iota == ik, inf, d)

    r0 = 1.0 / (sqs[0] + 1e-8)
    r1 = 1.0 / (sqs[1] + 1e-8)
    r2 = 1.0 / (sqs[2] + 1e-8)
    norm = (r0 + r1) + r2
    wt_ref[...] = jnp.concatenate([r0 / norm, r1 / norm, r2 / norm], axis=1)
    idx_ref[...] = jnp.concatenate(
        [idxs[0] + base, idxs[1] + base, idxs[2] + base], axis=1)


def _knn_b(xyz_b, centers_t_b, base):
    return pl.pallas_call(
        lambda *refs: _knn_body_b(base, *refs),
        grid=(_N // _NBLK_KNN,),
        in_specs=[
            pl.BlockSpec((_NBLK_KNN, 3), lambda j: (j, 0)),
            pl.BlockSpec((3, _M), lambda j: (0, 0)),
        ],
        out_specs=[
            pl.BlockSpec((_NBLK_KNN, 3), lambda j: (j, 0)),
            pl.BlockSpec((_NBLK_KNN, 3), lambda j: (j, 0)),
        ],
        out_shape=[
            jax.ShapeDtypeStruct((_N, 3), jnp.int32),
            jax.ShapeDtypeStruct((_N, 3), jnp.float32),
        ],
    )(xyz_b, centers_t_b)


_NW = 32
_G = _N * 3
_CHUNK = 128
_NCH = _G // (_NW * _CHUNK)


def _sc_gather_body(table_hbm, idx_hbm, out_hbm, idx_v, buf0, buf1, sem0, sem1):
    wid = lax.axis_index("s") * 2 + lax.axis_index("c")
    pltpu.sync_copy(idx_hbm.at[wid], idx_v)
    bufs = (buf0, buf1)
    sems = (sem0, sem1)
    copies = [None, None]
    copies[0] = pltpu.async_copy(table_hbm.at[idx_v.at[0]], buf0, sem0)
    for i in range(_NCH):
        if i + 1 < _NCH:
            copies[(i + 1) % 2] = pltpu.async_copy(
                table_hbm.at[idx_v.at[i + 1]], bufs[(i + 1) % 2],
                sems[(i + 1) % 2])
        copies[i % 2].wait()
        base = (wid * _NCH + i) * _CHUNK
        pltpu.sync_copy(bufs[i % 2], out_hbm.at[pl.ds(base, _CHUNK)])


def _sc_gather(table, idx_flat):
    mesh = plsc.VectorSubcoreMesh(core_axis_name="c", subcore_axis_name="s")
    k = pl.kernel(
        _sc_gather_body,
        out_type=jax.ShapeDtypeStruct((_G, _ROW), jnp.float32),
        mesh=mesh,
        scratch_types=[
            pltpu.VMEM((_NCH, _CHUNK), jnp.int32),
            pltpu.VMEM((_CHUNK, _ROW), jnp.float32),
            pltpu.VMEM((_CHUNK, _ROW), jnp.float32),
            pltpu.SemaphoreType.DMA,
            pltpu.SemaphoreType.DMA,
        ],
    )
    return k(table, idx_flat.reshape(_NW, _NCH, _CHUNK))


_NBLK_MLP = 512


def _gelu(x):
    return x * 0.5 * (1.0 + lax.erf(x * jnp.float32(0.7071067811865476)))


def _layer_norm(x, g, b):
    mu = jnp.mean(x, axis=-1, keepdims=True)
    var = jnp.mean((x - mu) ** 2, axis=-1, keepdims=True)
    return (x - mu) / jnp.sqrt(var + 1e-5) * g + b


def _dot_t(x, w):
    return lax.dot_general(x, w, (((1,), (1,)), ((), ())),
                           preferred_element_type=jnp.float32)


def _dot(x, w):
    return lax.dot_general(x, w, (((1,), (0,)), ((), ())),
                           preferred_element_type=jnp.float32)


def _mlp_body(g_ref, xyz_ref, rgb_ref, wt_ref,
              rW1t_ref, rb1_ref, rg_ref, rbt_ref, rW2_ref, rb2_ref,
              mW1_ref, mb1_ref, mg_ref, mbt_ref, mW2_ref, mb2_ref,
              fW1_ref, fb1_ref, fg_ref, fbt_ref, fW2_ref, fb2_ref,
              out_ref):
    g3 = g_ref[...]
    x = xyz_ref[...]
    wt = wt_ref[...]
    rW1t = rW1t_ref[...]
    skip = None
    for k in range(3):
        gk = g3[k]
        rela = gk[:, 0:3] - x
        h = _dot(rela, rW1t) + rb1_ref[...]
        h = _layer_norm(h, rg_ref[...], rbt_ref[...])
        h = _gelu(h)
        rf = _dot_t(h, rW2_ref[...]) + rb2_ref[...]
        contrib = (rf + gk[:, 3:3 + _FEATS]) * wt[:, k:k + 1]
        skip = contrib if skip is None else skip + contrib

    xf = jnp.concatenate([skip, rgb_ref[...]], axis=1)
    h2 = _dot_t(xf, mW1_ref[...]) + mb1_ref[...]
    h2 = _layer_norm(h2, mg_ref[...], mbt_ref[...])
    h2 = _gelu(h2)
    xf = _dot_t(h2, mW2_ref[...]) + mb2_ref[...]

    h3 = _dot_t(skip + xf, fW1_ref[...]) + fb1_ref[...]
    h3 = _layer_norm(h3, fg_ref[...], fbt_ref[...])
    h3 = _gelu(h3)
    out_ref[...] = _dot_t(h3, fW2_ref[...]) + fb2_ref[...]


def _mlps_b(gathered, xyz_b, rgb_b, wt_b, params):
    nj = _N // _NBLK_MLP

    def full(a):
        r = len(a.shape)
        return pl.BlockSpec(a.shape, lambda j: (0,) * r)

    return pl.pallas_call(
        _mlp_body,
        grid=(nj,),
        in_specs=[
            pl.BlockSpec((3, _NBLK_MLP, _ROW), lambda j: (0, j, 0)),
            pl.BlockSpec((_NBLK_MLP, 3), lambda j: (j, 0)),
            pl.BlockSpec((_NBLK_MLP, 3), lambda j: (j, 0)),
            pl.BlockSpec((_NBLK_MLP, 3), lambda j: (j, 0)),
        ] + [full(p) for p in params],
        out_specs=pl.BlockSpec((_NBLK_MLP, _FEATS), lambda j: (j, 0)),
        out_shape=jax.ShapeDtypeStruct((_N, _FEATS), jnp.float32),
    )(gathered, xyz_b, rgb_b, wt_b, *params)


def kernel(xyz, rgb, centers, center_feats, rW1, rb1, rg, rbt, rW2, rb2,
           mW1, mb1, mg, mbt, mW2, mb2, fW1, fb1, fg, fbt, fW2, fb2):
    table = jnp.concatenate(
        [centers.reshape(_B * _M, 3),
         center_feats.reshape(_B * _M, _FEATS),
         jnp.zeros((_B * _M, _ROW - 3 - _FEATS), jnp.float32)], axis=1)
    params = (rW1.T, rb1, rg, rbt, rW2, rb2, mW1, mb1, mg, mbt, mW2, mb2,
              fW1, fb1, fg, fbt, fW2, fb2)

    outs = []
    for b in range(_B):
        idx_b, wt_b = _knn_b(xyz[b], centers[b].T, b * _M)
        gathered = _sc_gather(table, idx_b.T.reshape(-1))
        gathered = gathered.reshape(3, _N, _ROW)
        outs.append(_mlps_b(gathered, xyz[b], rgb[b], wt_b, params))
    return jnp.stack(outs)


# speedup vs baseline: 26.3412x; 1.0111x over previous
import jax
import jax.numpy as jnp
from jax import lax
from jax.experimental import pallas as pl
from jax.experimental.pallas import tpu as pltpu
from jax.experimental.pallas import tpu_sc as plsc

_B, _N, _M = 2, 8192, 2048
_FEATS, _HIDDEN = 64, 128
_ROW = 128


_NBLK_KNN = 1024


def _knn_body(xyz_ref, ct_ref, c_ref, cf_ref, idx_ref, wt_ref, table_ref):
    b = pl.program_id(0)
    j = pl.program_id(1)

    @pl.when(j == 0)
    def _build_table():
        table_ref[:, 0:3] = c_ref[0]
        table_ref[:, 3:3 + _FEATS] = cf_ref[0]
        table_ref[:, 3 + _FEATS:] = jnp.zeros(
            (_M, _ROW - 3 - _FEATS), jnp.float32)

    x = xyz_ref[0]
    ct = ct_ref[0]
    d0 = x[:, 0:1] - ct[0:1, :]
    d1 = x[:, 1:2] - ct[1:2, :]
    d2 = x[:, 2:3] - ct[2:3, :]
    d = d0 * d0 + d1 * d1 + d2 * d2

    iota = lax.broadcasted_iota(jnp.int32, (_NBLK_KNN, _M), 1).astype(jnp.float32)
    inf = jnp.float32(jnp.inf)
    idxs = []
    sqs = []
    for k in range(3):
        mval = jnp.min(d, axis=1, keepdims=True)
        ik = jnp.min(jnp.where(d == mval, iota, jnp.float32(_M)),
                     axis=1, keepdims=True)
        idxs.append(ik.astype(jnp.int32))
        sqs.append(mval)
        if k < 2:
            d = jnp.where(iota == ik, inf, d)

    r0 = 1.0 / (sqs[0] + 1e-8)
    r1 = 1.0 / (sqs[1] + 1e-8)
    r2 = 1.0 / (sqs[2] + 1e-8)
    norm = (r0 + r1) + r2
    wt_ref[0] = jnp.concatenate([r0 / norm, r1 / norm, r2 / norm], axis=1)
    base = b * _M
    rows = [jnp.transpose(ik + base)[None, None] for ik in idxs]
    idx_ref[...] = jnp.concatenate(rows, axis=0)


def _knn(xyz, centers_t, centers, center_feats):
    nj = _N // _NBLK_KNN
    grid = (_B, nj)
    return pl.pallas_call(
        _knn_body,
        grid=grid,
        in_specs=[
            pl.BlockSpec((1, _NBLK_KNN, 3), lambda b, j: (b, j, 0)),
            pl.BlockSpec((1, 3, _M), lambda b, j: (b, 0, 0)),
            pl.BlockSpec((1, _M, 3), lambda b, j: (b, 0, 0)),
            pl.BlockSpec((1, _M, _FEATS), lambda b, j: (b, 0, 0)),
        ],
        out_specs=[
            pl.BlockSpec((3, 1, 1, _NBLK_KNN), lambda b, j: (0, b * nj + j, 0, 0)),
            pl.BlockSpec((1, _NBLK_KNN, 3), lambda b, j: (b, j, 0)),
            pl.BlockSpec((_M, _ROW), lambda b, j: (b, 0)),
        ],
        out_shape=[
            jax.ShapeDtypeStruct((3, _B * nj, 1, _NBLK_KNN), jnp.int32),
            jax.ShapeDtypeStruct((_B, _N, 3), jnp.float32),
            jax.ShapeDtypeStruct((_B * _M, _ROW), jnp.float32),
        ],
    )(xyz, centers_t, centers, center_feats)


_NW = 32
_G = _B * _N * 3
_CHUNK = 128
_NCH = _G // (_NW * _CHUNK)


def _sc_gather_body(table_hbm, idx_hbm, out_hbm, idx_v, buf0, buf1, sem0, sem1):
    wid = lax.axis_index("s") * 2 + lax.axis_index("c")
    pltpu.sync_copy(idx_hbm.at[wid], idx_v)
    bufs = (buf0, buf1)
    sems = (sem0, sem1)
    copies = [None, None]
    copies[0] = pltpu.async_copy(table_hbm.at[idx_v.at[0]], buf0, sem0)
    for i in range(_NCH):
        if i + 1 < _NCH:
            copies[(i + 1) % 2] = pltpu.async_copy(
                table_hbm.at[idx_v.at[i + 1]], bufs[(i + 1) % 2],
                sems[(i + 1) % 2])
        copies[i % 2].wait()
        base = (wid * _NCH + i) * _CHUNK
        pltpu.sync_copy(bufs[i % 2], out_hbm.at[pl.ds(base, _CHUNK)])


def _sc_gather(table, idx_flat):
    mesh = plsc.VectorSubcoreMesh(core_axis_name="c", subcore_axis_name="s")
    k = pl.kernel(
        _sc_gather_body,
        out_type=jax.ShapeDtypeStruct((_G, _ROW), jnp.float32),
        mesh=mesh,
        scratch_types=[
            pltpu.VMEM((_NCH, _CHUNK), jnp.int32),
            pltpu.VMEM((_CHUNK, _ROW), jnp.float32),
            pltpu.VMEM((_CHUNK, _ROW), jnp.float32),
            pltpu.SemaphoreType.DMA,
            pltpu.SemaphoreType.DMA,
        ],
    )
    return k(table, idx_flat.reshape(_NW, _NCH, _CHUNK))


_NBLK_MLP = 512


def _gelu(x):
    return x * 0.5 * (1.0 + lax.erf(x * jnp.float32(0.7071067811865476)))


def _layer_norm(x, g, b):
    mu = jnp.mean(x, axis=-1, keepdims=True)
    var = jnp.mean((x - mu) ** 2, axis=-1, keepdims=True)
    return (x - mu) / jnp.sqrt(var + 1e-5) * g + b


def _dot_t(x, w):
    return lax.dot_general(x, w, (((1,), (1,)), ((), ())),
                           preferred_element_type=jnp.float32)


def _dot(x, w):
    return lax.dot_general(x, w, (((1,), (0,)), ((), ())),
                           preferred_element_type=jnp.float32)


def _mlp_body(g_ref, xyz_ref, rgb_ref, wt_ref,
              rW1t_ref, rb1_ref, rg_ref, rbt_ref, rW2_ref, rb2_ref,
              mW1_ref, mb1_ref, mg_ref, mbt_ref, mW2_ref, mb2_ref,
              fW1_ref, fb1_ref, fg_ref, fbt_ref, fW2_ref, fb2_ref,
              out_ref):
    g3 = g_ref[...]
    x = xyz_ref[0]
    wt = wt_ref[0]
    rW1t = rW1t_ref[...]
    skip = None
    for k in range(3):
        gk = g3[k]
        rela = gk[:, 0:3] - x
        h = _dot(rela, rW1t) + rb1_ref[...]
        h = _layer_norm(h, rg_ref[...], rbt_ref[...])
        h = _gelu(h)
        rf = _dot_t(h, rW2_ref[...]) + rb2_ref[...]
        contrib = (rf + gk[:, 3:3 + _FEATS]) * wt[:, k:k + 1]
        skip = contrib if skip is None else skip + contrib

    xf = jnp.concatenate([skip, rgb_ref[0]], axis=1)
    h2 = _dot_t(xf, mW1_ref[...]) + mb1_ref[...]
    h2 = _layer_norm(h2, mg_ref[...], mbt_ref[...])
    h2 = _gelu(h2)
    xf = _dot_t(h2, mW2_ref[...]) + mb2_ref[...]

    h3 = _dot_t(skip + xf, fW1_ref[...]) + fb1_ref[...]
    h3 = _layer_norm(h3, fg_ref[...], fbt_ref[...])
    h3 = _gelu(h3)
    out_ref[0] = _dot_t(h3, fW2_ref[...]) + fb2_ref[...]


def _mlps(gathered, xyz, rgb, wt, params):
    nj = _N // _NBLK_MLP
    grid = (_B, nj)

    def full(a):
        r = len(a.shape)
        return pl.BlockSpec(a.shape, lambda b, j: (0,) * r)

    return pl.pallas_call(
        _mlp_body,
        grid=grid,
        in_specs=[
            pl.BlockSpec((3, _NBLK_MLP, _ROW), lambda b, j: (0, b * nj + j, 0)),
            pl.BlockSpec((1, _NBLK_MLP, 3), lambda b, j: (b, j, 0)),
            pl.BlockSpec((1, _NBLK_MLP, 3), lambda b, j: (b, j, 0)),
            pl.BlockSpec((1, _NBLK_MLP, 3), lambda b, j: (b, j, 0)),
        ] + [full(p) for p in params],
        out_specs=pl.BlockSpec((1, _NBLK_MLP, _FEATS), lambda b, j: (b, j, 0)),
        out_shape=jax.ShapeDtypeStruct((_B, _N, _FEATS), jnp.float32),
    )(gathered, xyz, rgb, wt, *params)


def kernel(xyz, rgb, centers, center_feats, rW1, rb1, rg, rbt, rW2, rb2,
           mW1, mb1, mg, mbt, mW2, mb2, fW1, fb1, fg, fbt, fW2, fb2):
    idx, wt, table = _knn(xyz, centers.transpose(0, 2, 1), centers,
                          center_feats)
    gathered = _sc_gather(table, idx.reshape(-1))
    gathered = gathered.reshape(3, _B * _N, _ROW)
    params = (rW1.T, rb1, rg, rbt, rW2, rb2, mW1, mb1, mg, mbt, mW2, mb2,
              fW1, fb1, fg, fbt, fW2, fb2)
    return _mlps(gathered, xyz, rgb, wt, params)
